# Initial kernel scaffold; baseline (speedup 1.0000x reference)
import jax
import jax.numpy as jnp
from jax.experimental import pallas as pl


def kernel(x, edge_index, batch, W1, b1, W2, b2, W3, b3, bn_gamma, bn_beta, bn_mean, bn_var, lin_W, lin_b):
    raise NotImplementedError("write your pallas kernel here")


# speedup vs baseline: 11.3578x; 11.3578x over previous
import functools

import jax
import jax.numpy as jnp
from jax import lax
from jax.experimental import pallas as pl
from jax.experimental.pallas import tpu as pltpu
from jax.experimental.pallas import tpu_sc as plsc

_N = 10000
_E = 320000
_F = 128
_H = 64
_C = 2
_G = 64

_NC = 2
_NS = 16
_NW = _NC * _NS
_CHUNK = 128
_NP = 10240
_RPT = _NP // _NS
_NCHUNKS = 80
_EPT = _NCHUNKS * _CHUNK
_EP = _EPT * _NW
_BLK = 512
_NBLK = _NP // _BLK


def _make_sc_agg():
    mesh = plsc.VectorSubcoreMesh(core_axis_name="c", subcore_axis_name="s")

    @functools.partial(
        pl.kernel,
        out_type=jax.ShapeDtypeStruct((_NC, _NP, _H), jnp.float32),
        mesh=mesh,
        compiler_params=pltpu.CompilerParams(use_tc_tiling_on_sc=False),
        scratch_types=[
            pltpu.VMEM((_NCHUNKS, _CHUNK), jnp.int32),
            pltpu.VMEM((_NCHUNKS, _CHUNK), jnp.int32),
            pltpu.VMEM((_CHUNK, _H), jnp.float32),
            pltpu.VMEM((_CHUNK, _H), jnp.float32),
            pltpu.VMEM_SHARED((_NP, _H), jnp.float32),
            pltpu.SemaphoreType.DMA,
            pltpu.SemaphoreType.DMA,
        ],
    )
    def agg(y_hbm, src_hbm, dst_hbm, zeros_hbm, out_hbm,
            src_v, dst_v, rows_a, rows_b, acc, sem_a, sem_b):
        cid = lax.axis_index("c")
        sid = lax.axis_index("s")
        wid = cid * _NS + sid
        pltpu.sync_copy(zeros_hbm, acc.at[pl.ds(sid * _RPT, _RPT)])
        pltpu.sync_copy(src_hbm.at[wid], src_v)
        pltpu.sync_copy(dst_hbm.at[wid], dst_v)
        plsc.subcore_barrier()

        pltpu.async_copy(y_hbm.at[src_v.at[0]], rows_a, sem_a)

        def pair(p, carry):
            j = p * 2
            pltpu.make_async_copy(y_hbm.at[src_v.at[j]], rows_a, sem_a).wait()
            pltpu.async_copy(y_hbm.at[src_v.at[j + 1]], rows_b, sem_b)
            pltpu.sync_copy(rows_a, acc.at[dst_v.at[j]], add=True)
            pltpu.make_async_copy(y_hbm.at[src_v.at[j + 1]], rows_b, sem_b).wait()

            @pl.when(p + 1 < _NCHUNKS // 2)
            def _():
                pltpu.async_copy(y_hbm.at[src_v.at[j + 2]], rows_a, sem_a)

            pltpu.sync_copy(rows_b, acc.at[dst_v.at[j + 1]], add=True)
            return carry

        lax.fori_loop(0, _NCHUNKS // 2, pair, 0)
        plsc.subcore_barrier()
        pltpu.sync_copy(acc.at[pl.ds(sid * _RPT, _RPT)],
                        out_hbm.at[cid, pl.ds(sid * _RPT, _RPT)])

    return agg


def _make_sc_deg():
    mesh = plsc.VectorSubcoreMesh(core_axis_name="c", subcore_axis_name="s")

    @functools.partial(
        pl.kernel,
        out_type=jax.ShapeDtypeStruct((_NC, _NP, 16), jnp.float32),
        mesh=mesh,
        compiler_params=pltpu.CompilerParams(use_tc_tiling_on_sc=False),
        scratch_types=[
            pltpu.VMEM((_NCHUNKS, _CHUNK), jnp.int32),
            pltpu.VMEM((_CHUNK, 16), jnp.float32),
            pltpu.VMEM_SHARED((_NP, 16), jnp.float32),
        ],
    )
    def deg(dst_hbm, ones_hbm, zeros_hbm, out_hbm, dst_v, ones_v, acc):
        cid = lax.axis_index("c")
        sid = lax.axis_index("s")
        wid = cid * _NS + sid
        pltpu.sync_copy(zeros_hbm, acc.at[pl.ds(sid * _RPT, _RPT)])
        pltpu.sync_copy(dst_hbm.at[wid], dst_v)
        pltpu.sync_copy(ones_hbm, ones_v)
        plsc.subcore_barrier()

        def body(j, carry):
            pltpu.sync_copy(ones_v, acc.at[dst_v.at[j]], add=True)
            return carry

        lax.fori_loop(0, _NCHUNKS, body, 0)
        plsc.subcore_barrier()
        pltpu.sync_copy(acc.at[pl.ds(sid * _RPT, _RPT)],
                        out_hbm.at[cid, pl.ds(sid * _RPT, _RPT)])

    return deg


@functools.cache
def _get_sc_agg():
    return _make_sc_agg()


@functools.cache
def _get_sc_deg():
    return _make_sc_deg()


def _mm1_body(x_ref, w_ref, o_ref):
    o_ref[...] = jnp.dot(x_ref[...], w_ref[...],
                         preferred_element_type=jnp.float32)


_mm1 = pl.pallas_call(
    _mm1_body,
    grid=(_NBLK,),
    in_specs=[pl.BlockSpec((_BLK, _F), lambda i: (i, 0)),
              pl.BlockSpec((_F, _H), lambda i: (0, 0))],
    out_specs=pl.BlockSpec((_BLK, _H), lambda i: (i, 0)),
    out_shape=jax.ShapeDtypeStruct((_NP, _H), jnp.float32),
)


def _dinv_body(degp_ref, xw_ref, dinv_ref, y_ref):
    deg = degp_ref[0, :, 0:1] + degp_ref[1, :, 0:1] + 1.0
    dinv = lax.rsqrt(deg)
    dinv_ref[...] = dinv
    y_ref[...] = xw_ref[...] * dinv


_dinv_k = pl.pallas_call(
    _dinv_body,
    grid=(_NBLK,),
    in_specs=[pl.BlockSpec((_NC, _BLK, 16), lambda i: (0, i, 0)),
              pl.BlockSpec((_BLK, _H), lambda i: (i, 0))],
    out_specs=[pl.BlockSpec((_BLK, 1), lambda i: (i, 0)),
               pl.BlockSpec((_BLK, _H), lambda i: (i, 0))],
    out_shape=[jax.ShapeDtypeStruct((_NP, 1), jnp.float32),
               jax.ShapeDtypeStruct((_NP, _H), jnp.float32)],
)


def _mid_body(aggp_ref, y_ref, dinv_ref, b_ref, w_ref, o_ref):
    dinv = dinv_ref[...]
    h = aggp_ref[0] + aggp_ref[1] + y_ref[...]
    h = jnp.maximum(h * dinv + b_ref[...], 0.0)
    o_ref[...] = jnp.dot(h * dinv, w_ref[...],
                         preferred_element_type=jnp.float32)


_mid_k = pl.pallas_call(
    _mid_body,
    grid=(_NBLK,),
    in_specs=[pl.BlockSpec((_NC, _BLK, _H), lambda i: (0, i, 0)),
              pl.BlockSpec((_BLK, _H), lambda i: (i, 0)),
              pl.BlockSpec((_BLK, 1), lambda i: (i, 0)),
              pl.BlockSpec((1, _H), lambda i: (0, 0)),
              pl.BlockSpec((_H, _H), lambda i: (0, 0))],
    out_specs=pl.BlockSpec((_BLK, _H), lambda i: (i, 0)),
    out_shape=jax.ShapeDtypeStruct((_NP, _H), jnp.float32),
)


def _pool_body(aggp_ref, y_ref, dinv_ref, b_ref, batch_ref, sums_ref, cnt_ref):
    i = pl.program_id(0)
    dinv = dinv_ref[...]
    h = aggp_ref[0] + aggp_ref[1] + y_ref[...]
    h = jnp.maximum(h * dinv + b_ref[...], 0.0)
    onehot = (batch_ref[...] ==
              lax.broadcasted_iota(jnp.int32, (_BLK, _G), 1)).astype(jnp.float32)
    dn = (((0,), (0,)), ((), ()))
    sums_part = lax.dot_general(onehot, h, dn,
                                preferred_element_type=jnp.float32)
    cnt_part = lax.dot_general(onehot, jnp.ones((_BLK, 1), jnp.float32), dn,
                               preferred_element_type=jnp.float32)

    @pl.when(i == 0)
    def _():
        sums_ref[...] = sums_part
        cnt_ref[...] = cnt_part

    @pl.when(i > 0)
    def _():
        sums_ref[...] += sums_part
        cnt_ref[...] += cnt_part


_pool_k = pl.pallas_call(
    _pool_body,
    grid=(_NBLK,),
    in_specs=[pl.BlockSpec((_NC, _BLK, _H), lambda i: (0, i, 0)),
              pl.BlockSpec((_BLK, _H), lambda i: (i, 0)),
              pl.BlockSpec((_BLK, 1), lambda i: (i, 0)),
              pl.BlockSpec((1, _H), lambda i: (0, 0)),
              pl.BlockSpec((_BLK, 1), lambda i: (i, 0))],
    out_specs=[pl.BlockSpec((_G, _H), lambda i: (0, 0)),
               pl.BlockSpec((_G, 1), lambda i: (0, 0))],
    out_shape=[jax.ShapeDtypeStruct((_G, _H), jnp.float32),
               jax.ShapeDtypeStruct((_G, 1), jnp.float32)],
)


def _head_body(sums_ref, cnt_ref, gamma_ref, beta_ref, mean_ref, var_ref,
               lw_ref, lb_ref, xbn_ref, out_ref):
    cnt = jnp.maximum(cnt_ref[...], 1.0)
    pooled = sums_ref[...] / cnt
    scale = lax.rsqrt(var_ref[...] + 1e-5) * gamma_ref[...]
    xbn = (pooled - mean_ref[...]) * scale + beta_ref[...]
    xbn_ref[...] = xbn
    out_ref[...] = jnp.maximum(
        jnp.dot(xbn, lw_ref[...], preferred_element_type=jnp.float32)
        + lb_ref[...], 0.0)


_head_k = pl.pallas_call(
    _head_body,
    out_shape=[jax.ShapeDtypeStruct((_G, _H), jnp.float32),
               jax.ShapeDtypeStruct((_G, _F), jnp.float32)],
)


def kernel(x, edge_index, batch, W1, b1, W2, b2, W3, b3,
           bn_gamma, bn_beta, bn_mean, bn_var, lin_W, lin_b):
    xp = jnp.pad(x, ((0, _NP - _N), (0, 0)))
    src = jnp.pad(edge_index[0], (0, _EP - _E)).reshape(_NW, _NCHUNKS, _CHUNK)
    dst = jnp.pad(edge_index[1], (0, _EP - _E),
                  constant_values=_N).reshape(_NW, _NCHUNKS, _CHUNK)
    batch_p = jnp.pad(batch, (0, _NP - _N),
                      constant_values=_G).reshape(_NP, 1)
    zeros_h = jnp.zeros((_RPT, _H), jnp.float32)
    zeros16 = jnp.zeros((_RPT, 16), jnp.float32)
    ones16 = jnp.ones((_CHUNK, 16), jnp.float32)
    lw_p = jnp.pad(lin_W, ((0, 0), (0, _F - _C)))
    lb_p = jnp.pad(lin_b, (0, _F - _C)).reshape(1, _F)

    sc_agg = _get_sc_agg()
    sc_deg = _get_sc_deg()

    xw1 = _mm1(xp, W1)
    degp = sc_deg(dst, ones16, zeros16)
    dinv, y1 = _dinv_k(degp, xw1)

    aggp1 = sc_agg(y1, src, dst, zeros_h)
    y2 = _mid_k(aggp1, y1, dinv, b1.reshape(1, _H), W2)
    aggp2 = sc_agg(y2, src, dst, zeros_h)
    y3 = _mid_k(aggp2, y2, dinv, b2.reshape(1, _H), W3)
    aggp3 = sc_agg(y3, src, dst, zeros_h)

    sums, cnt = _pool_k(aggp3, y3, dinv, b3.reshape(1, _H), batch_p)
    xbn, out_p = _head_k(sums, cnt, bn_gamma.reshape(1, _H),
                         bn_beta.reshape(1, _H), bn_mean.reshape(1, _H),
                         bn_var.reshape(1, _H), lw_p, lb_p)
    return (xbn, out_p[:, :_C])


# speedup vs baseline: 11.8422x; 1.0427x over previous
import functools

import jax
import jax.numpy as jnp
from jax import lax
from jax.experimental import pallas as pl
from jax.experimental.pallas import tpu as pltpu
from jax.experimental.pallas import tpu_sc as plsc

_N = 10000
_E = 320000
_F = 128
_H = 64
_C = 2
_G = 64

_NC = 2
_NS = 16
_NW = _NC * _NS
_CHUNK = 128
_NP = 10240
_RPT = _NP // _NS
_NCHUNKS = 80
_EPT = _NCHUNKS * _CHUNK
_EP = _EPT * _NW
_BLK = 512
_NBLK = _NP // _BLK
_DEPTH = 4


def _make_sc_agg():
    mesh = plsc.VectorSubcoreMesh(core_axis_name="c", subcore_axis_name="s")

    @functools.partial(
        pl.kernel,
        out_type=jax.ShapeDtypeStruct((_NC, _NP, _H), jnp.float32),
        mesh=mesh,
        compiler_params=pltpu.CompilerParams(use_tc_tiling_on_sc=False),
        scratch_types=[
            pltpu.VMEM((_NCHUNKS, _CHUNK), jnp.int32),
            pltpu.VMEM((_NCHUNKS, _CHUNK), jnp.int32),
            [pltpu.VMEM((_CHUNK, _H), jnp.float32)] * _DEPTH,
            [pltpu.SemaphoreType.DMA] * _DEPTH,
            [pltpu.SemaphoreType.DMA] * _DEPTH,
            pltpu.VMEM_SHARED((_NP, _H), jnp.float32),
        ],
    )
    def agg(y_hbm, src_hbm, dst_hbm, zeros_hbm, out_hbm,
            src_v, dst_v, rows, gsem, ssem, acc):
        cid = lax.axis_index("c")
        sid = lax.axis_index("s")
        wid = cid * _NS + sid
        pltpu.sync_copy(zeros_hbm, acc.at[pl.ds(sid * _RPT, _RPT)])
        pltpu.sync_copy(src_hbm.at[wid], src_v)
        pltpu.sync_copy(dst_hbm.at[wid], dst_v)
        plsc.subcore_barrier()

        for b in range(_DEPTH):
            pltpu.async_copy(y_hbm.at[src_v.at[b]], rows[b], gsem[b])

        def group(g, carry):
            j0 = g * _DEPTH
            for b in range(_DEPTH):
                j = j0 + b
                pltpu.make_async_copy(y_hbm.at[src_v.at[j]], rows[b],
                                      gsem[b]).wait()
                pltpu.async_copy(rows[b], acc.at[dst_v.at[j]], ssem[b],
                                 add=True)
            for b in range(_DEPTH):
                j = j0 + b
                pltpu.make_async_copy(rows[b], acc.at[dst_v.at[j]],
                                      ssem[b]).wait()

                @pl.when(g + 1 < _NCHUNKS // _DEPTH)
                def _():
                    pltpu.async_copy(y_hbm.at[src_v.at[j + _DEPTH]], rows[b],
                                     gsem[b])

            return carry

        lax.fori_loop(0, _NCHUNKS // _DEPTH, group, 0)
        plsc.subcore_barrier()
        pltpu.sync_copy(acc.at[pl.ds(sid * _RPT, _RPT)],
                        out_hbm.at[cid, pl.ds(sid * _RPT, _RPT)])

    return agg


def _make_sc_deg():
    mesh = plsc.VectorSubcoreMesh(core_axis_name="c", subcore_axis_name="s")

    @functools.partial(
        pl.kernel,
        out_type=jax.ShapeDtypeStruct((_NC, _NP, 16), jnp.float32),
        mesh=mesh,
        compiler_params=pltpu.CompilerParams(use_tc_tiling_on_sc=False),
        scratch_types=[
            pltpu.VMEM((_NCHUNKS, _CHUNK), jnp.int32),
            pltpu.VMEM((_CHUNK, 16), jnp.float32),
            pltpu.VMEM_SHARED((_NP, 16), jnp.float32),
        ],
    )
    def deg(dst_hbm, ones_hbm, zeros_hbm, out_hbm, dst_v, ones_v, acc):
        cid = lax.axis_index("c")
        sid = lax.axis_index("s")
        wid = cid * _NS + sid
        pltpu.sync_copy(zeros_hbm, acc.at[pl.ds(sid * _RPT, _RPT)])
        pltpu.sync_copy(dst_hbm.at[wid], dst_v)
        pltpu.sync_copy(ones_hbm, ones_v)
        plsc.subcore_barrier()

        def body(j, carry):
            pltpu.sync_copy(ones_v, acc.at[dst_v.at[j]], add=True)
            return carry

        lax.fori_loop(0, _NCHUNKS, body, 0)
        plsc.subcore_barrier()
        pltpu.sync_copy(acc.at[pl.ds(sid * _RPT, _RPT)],
                        out_hbm.at[cid, pl.ds(sid * _RPT, _RPT)])

    return deg


@functools.cache
def _get_sc_agg():
    return _make_sc_agg()


@functools.cache
def _get_sc_deg():
    return _make_sc_deg()


def _mm1_body(x_ref, w_ref, o_ref):
    o_ref[...] = jnp.dot(x_ref[...], w_ref[...],
                         preferred_element_type=jnp.float32)


_mm1 = pl.pallas_call(
    _mm1_body,
    grid=(_NBLK,),
    in_specs=[pl.BlockSpec((_BLK, _F), lambda i: (i, 0)),
              pl.BlockSpec((_F, _H), lambda i: (0, 0))],
    out_specs=pl.BlockSpec((_BLK, _H), lambda i: (i, 0)),
    out_shape=jax.ShapeDtypeStruct((_NP, _H), jnp.float32),
)


def _dinv_body(degp_ref, xw_ref, dinv_ref, y_ref):
    deg = degp_ref[0, :, 0:1] + degp_ref[1, :, 0:1] + 1.0
    dinv = lax.rsqrt(deg)
    dinv_ref[...] = dinv
    y_ref[...] = xw_ref[...] * dinv


_dinv_k = pl.pallas_call(
    _dinv_body,
    grid=(_NBLK,),
    in_specs=[pl.BlockSpec((_NC, _BLK, 16), lambda i: (0, i, 0)),
              pl.BlockSpec((_BLK, _H), lambda i: (i, 0))],
    out_specs=[pl.BlockSpec((_BLK, 1), lambda i: (i, 0)),
               pl.BlockSpec((_BLK, _H), lambda i: (i, 0))],
    out_shape=[jax.ShapeDtypeStruct((_NP, 1), jnp.float32),
               jax.ShapeDtypeStruct((_NP, _H), jnp.float32)],
)


def _mid_body(aggp_ref, y_ref, dinv_ref, b_ref, w_ref, o_ref):
    dinv = dinv_ref[...]
    h = aggp_ref[0] + aggp_ref[1] + y_ref[...]
    h = jnp.maximum(h * dinv + b_ref[...], 0.0)
    o_ref[...] = jnp.dot(h * dinv, w_ref[...],
                         preferred_element_type=jnp.float32)


_mid_k = pl.pallas_call(
    _mid_body,
    grid=(_NBLK,),
    in_specs=[pl.BlockSpec((_NC, _BLK, _H), lambda i: (0, i, 0)),
              pl.BlockSpec((_BLK, _H), lambda i: (i, 0)),
              pl.BlockSpec((_BLK, 1), lambda i: (i, 0)),
              pl.BlockSpec((1, _H), lambda i: (0, 0)),
              pl.BlockSpec((_H, _H), lambda i: (0, 0))],
    out_specs=pl.BlockSpec((_BLK, _H), lambda i: (i, 0)),
    out_shape=jax.ShapeDtypeStruct((_NP, _H), jnp.float32),
)


def _pool_body(aggp_ref, y_ref, dinv_ref, b_ref, batch_ref, sums_ref, cnt_ref):
    i = pl.program_id(0)
    dinv = dinv_ref[...]
    h = aggp_ref[0] + aggp_ref[1] + y_ref[...]
    h = jnp.maximum(h * dinv + b_ref[...], 0.0)
    onehot = (batch_ref[...] ==
              lax.broadcasted_iota(jnp.int32, (_BLK, _G), 1)).astype(jnp.float32)
    dn = (((0,), (0,)), ((), ()))
    sums_part = lax.dot_general(onehot, h, dn,
                                preferred_element_type=jnp.float32)
    cnt_part = lax.dot_general(onehot, jnp.ones((_BLK, 1), jnp.float32), dn,
                               preferred_element_type=jnp.float32)

    @pl.when(i == 0)
    def _():
        sums_ref[...] = sums_part
        cnt_ref[...] = cnt_part

    @pl.when(i > 0)
    def _():
        sums_ref[...] += sums_part
        cnt_ref[...] += cnt_part


_pool_k = pl.pallas_call(
    _pool_body,
    grid=(_NBLK,),
    in_specs=[pl.BlockSpec((_NC, _BLK, _H), lambda i: (0, i, 0)),
              pl.BlockSpec((_BLK, _H), lambda i: (i, 0)),
              pl.BlockSpec((_BLK, 1), lambda i: (i, 0)),
              pl.BlockSpec((1, _H), lambda i: (0, 0)),
              pl.BlockSpec((_BLK, 1), lambda i: (i, 0))],
    out_specs=[pl.BlockSpec((_G, _H), lambda i: (0, 0)),
               pl.BlockSpec((_G, 1), lambda i: (0, 0))],
    out_shape=[jax.ShapeDtypeStruct((_G, _H), jnp.float32),
               jax.ShapeDtypeStruct((_G, 1), jnp.float32)],
)


def _head_body(sums_ref, cnt_ref, gamma_ref, beta_ref, mean_ref, var_ref,
               lw_ref, lb_ref, xbn_ref, out_ref):
    cnt = jnp.maximum(cnt_ref[...], 1.0)
    pooled = sums_ref[...] / cnt
    scale = lax.rsqrt(var_ref[...] + 1e-5) * gamma_ref[...]
    xbn = (pooled - mean_ref[...]) * scale + beta_ref[...]
    xbn_ref[...] = xbn
    out_ref[...] = jnp.maximum(
        jnp.dot(xbn, lw_ref[...], preferred_element_type=jnp.float32)
        + lb_ref[...], 0.0)


_head_k = pl.pallas_call(
    _head_body,
    out_shape=[jax.ShapeDtypeStruct((_G, _H), jnp.float32),
               jax.ShapeDtypeStruct((_G, _F), jnp.float32)],
)


def kernel(x, edge_index, batch, W1, b1, W2, b2, W3, b3,
           bn_gamma, bn_beta, bn_mean, bn_var, lin_W, lin_b):
    xp = jnp.pad(x, ((0, _NP - _N), (0, 0)))
    src = jnp.pad(edge_index[0], (0, _EP - _E)).reshape(_NW, _NCHUNKS, _CHUNK)
    dst = jnp.pad(edge_index[1], (0, _EP - _E),
                  constant_values=_N).reshape(_NW, _NCHUNKS, _CHUNK)
    batch_p = jnp.pad(batch, (0, _NP - _N),
                      constant_values=_G).reshape(_NP, 1)
    zeros_h = jnp.zeros((_RPT, _H), jnp.float32)
    zeros16 = jnp.zeros((_RPT, 16), jnp.float32)
    ones16 = jnp.ones((_CHUNK, 16), jnp.float32)
    lw_p = jnp.pad(lin_W, ((0, 0), (0, _F - _C)))
    lb_p = jnp.pad(lin_b, (0, _F - _C)).reshape(1, _F)

    sc_agg = _get_sc_agg()
    sc_deg = _get_sc_deg()

    xw1 = _mm1(xp, W1)
    degp = sc_deg(dst, ones16, zeros16)
    dinv, y1 = _dinv_k(degp, xw1)

    aggp1 = sc_agg(y1, src, dst, zeros_h)
    y2 = _mid_k(aggp1, y1, dinv, b1.reshape(1, _H), W2)
    aggp2 = sc_agg(y2, src, dst, zeros_h)
    y3 = _mid_k(aggp2, y2, dinv, b2.reshape(1, _H), W3)
    aggp3 = sc_agg(y3, src, dst, zeros_h)

    sums, cnt = _pool_k(aggp3, y3, dinv, b3.reshape(1, _H), batch_p)
    xbn, out_p = _head_k(sums, cnt, bn_gamma.reshape(1, _H),
                         bn_beta.reshape(1, _H), bn_mean.reshape(1, _H),
                         bn_var.reshape(1, _H), lw_p, lb_p)
    return (xbn, out_p[:, :_C])
